# Initial kernel scaffold; baseline (speedup 1.0000x reference)
import jax
import jax.numpy as jnp
from jax.experimental import pallas as pl


def kernel(data, token_all_veh):
    raise NotImplementedError("write your pallas kernel here")


# speedup vs baseline: 8.3457x; 8.3457x over previous
import functools

import jax
import jax.numpy as jnp
from jax import lax
from jax.experimental import pallas as pl
from jax.experimental.pallas import tpu as pltpu
from jax.experimental.pallas import tpu_sc as plsc

N_TOK = 8192
N_ST = 18
N_TILE = 16
TOK_PER_TILE = N_TOK // N_TILE
VREGS = TOK_PER_TILE // 16

_PI = 3.14159265358979323846
_F32 = jnp.float32


def _prep_body(p8_ref, dpad_ref, feat_ref, g_ref):
    p = p8_ref[...]
    x0, y0, x1, y1 = p[0], p[1], p[2], p[3]
    x2, y2, x3, y3 = p[4], p[5], p[6], p[7]
    mx = (x0 + x1 + x2 + x3) * 0.25
    my = (y0 + y1 + y2 + y3) * 0.25
    dx = x0 - x3
    dy = y0 - y3
    a = jnp.arctan2(dy, dx)
    ca = jnp.cos(a)
    sa = jnp.sin(a)
    z = jnp.zeros_like(mx)
    feat_ref[...] = jnp.stack([mx, my, a, ca, sa, z, z, z], axis=0)

    d = dpad_ref[...]
    x = d[0]
    y = d[1]
    th = d[2]
    hc = 0.5 * jnp.cos(th)
    hs = 0.5 * jnp.sin(th)
    lc = 4.8 * hc
    ls = 4.8 * hs
    wc = 2.0 * hc
    ws = 2.0 * hs
    g_ref[...] = jnp.stack(
        [x + lc - ws, y + ls + wc,
         x + lc + ws, y + ls - wc,
         x - lc + ws, y - ls - wc,
         x - lc - ws, y - ls + wc],
        axis=0)


def _rsqrt(x):
    i = lax.bitcast_convert_type(x, jnp.int32)
    i = jnp.int32(0x5F3759DF) - lax.shift_right_logical(i, 1)
    y = lax.bitcast_convert_type(i, _F32)
    xh = x * _F32(0.5)
    for _ in range(3):
        y = y * (_F32(1.5) - xh * y * y)
    return y


def _sc_body(p_hbm, feat_hbm, g_hbm, out_hbm,
             p_v, g_v, out_v, cand_v, stage_v, featrow_v, sh_min, sh_feat):
    cid = lax.axis_index("c")
    sid = lax.axis_index("s")

    pltpu.sync_copy(p_hbm.at[sid], p_v)
    pltpu.sync_copy(g_hbm, g_v)
    pltpu.sync_copy(feat_hbm.at[pl.ds(sid * TOK_PER_TILE, TOK_PER_TILE)],
                    sh_feat.at[pl.ds(sid * TOK_PER_TILE, TOK_PER_TILE)])
    plsc.subcore_barrier()

    ii = jax.lax.broadcasted_iota(jnp.int32, (16,), 0)
    lanes = ii

    def step(i, carry):
        tx, ty, c, s, h = carry
        grow = g_v[i]
        gpx = []
        gpy = []
        for k in range(4):
            gx = grow[2 * k] - tx
            gy = grow[2 * k + 1] - ty
            gpx.append(jnp.full((16,), c * gx + s * gy, _F32))
            gpy.append(jnp.full((16,), c * gy - s * gx, _F32))

        def vbody(v, bc):
            bd, bi = bc
            base = v * 16
            dist = jnp.zeros((16,), _F32)
            for k in range(4):
                px = p_v[2 * k, pl.ds(base, 16)]
                py = p_v[2 * k + 1, pl.ds(base, 16)]
                dxv = px - gpx[k]
                dyv = py - gpy[k]
                d2 = jnp.maximum(dxv * dxv + dyv * dyv, _F32(1e-12))
                dist = dist + d2 * _rsqrt(d2)
            iv = jnp.full((16,), sid * TOK_PER_TILE + base, jnp.int32) + lanes
            m = dist < bd
            bd = jnp.where(m, dist, bd)
            bi = jnp.where(m, iv, bi)
            return bd, bi

        bd0 = jnp.full((16,), _F32(3.0e38))
        bi0 = jnp.zeros((16,), jnp.int32)
        bd, bi = lax.fori_loop(0, VREGS, vbody, (bd0, bi0))

        md = jnp.min(bd)
        li = jnp.min(jnp.where(bd == md, bi, jnp.int32(1 << 30)))

        stage_v[...] = jnp.where(ii == 0, jnp.full((16,), md),
                                 jnp.full((16,), li.astype(_F32)))
        pltpu.sync_copy(stage_v, sh_min.at[i, sid])
        plsc.subcore_barrier()
        pltpu.sync_copy(sh_min.at[i], cand_v)

        crow = cand_v[0]
        gbd = crow[0]
        gbi = crow[1]
        for t in range(1, N_TILE):
            crow = cand_v[t]
            dtv = crow[0]
            itv = crow[1]
            better = dtv < gbd
            gbd = jnp.where(better, dtv, gbd)
            gbi = jnp.where(better, itv, gbi)
        gidx = gbi.astype(jnp.int32)

        pltpu.sync_copy(sh_feat.at[gidx], featrow_v)
        fr = featrow_v[...]
        fmx = fr[0]
        fmy = fr[1]
        fa = fr[2]
        fca = fr[3]
        fsa = fr[4]

        tx2 = tx + c * fmx - s * fmy
        ty2 = ty + s * fmx + c * fmy
        c2 = c * fca - s * fsa
        s2 = s * fca + c * fsa
        h2 = h + fa
        h2 = jnp.where(h2 > _F32(_PI), h2 - _F32(2 * _PI), h2)
        h2 = jnp.where(h2 <= _F32(-_PI), h2 + _F32(2 * _PI), h2)

        out_v[i] = jnp.where(
            ii == 0, jnp.full((16,), tx2),
            jnp.where(ii == 1, jnp.full((16,), ty2),
                      jnp.where(ii == 2, jnp.full((16,), h2),
                                jnp.full((16,), gbi))))
        return tx2, ty2, c2, s2, h2

    carry0 = (_F32(0.0), _F32(0.0), _F32(1.0), _F32(0.0), _F32(0.0))
    lax.fori_loop(0, N_ST, step, carry0)

    @pl.when(jnp.logical_and(cid == 0, sid == 0))
    def _():
        pltpu.sync_copy(out_v, out_hbm)


_sc_loop = functools.partial(
    pl.kernel,
    out_type=jax.ShapeDtypeStruct((N_ST, 16), _F32),
    mesh=plsc.VectorSubcoreMesh(core_axis_name="c", subcore_axis_name="s",
                                num_cores=2, num_subcores=N_TILE),
    compiler_params=pltpu.CompilerParams(needs_layout_passes=False),
    scratch_types=[
        pltpu.VMEM((8, TOK_PER_TILE), _F32),
        pltpu.VMEM((N_ST, 16), _F32),
        pltpu.VMEM((N_ST, 16), _F32),
        pltpu.VMEM((N_TILE, 16), _F32),
        pltpu.VMEM((16,), _F32),
        pltpu.VMEM((16,), _F32),
        pltpu.VMEM_SHARED((N_ST, N_TILE, 16), _F32),
        pltpu.VMEM_SHARED((N_TOK, 16), _F32),
    ],
)(_sc_body)


def kernel(data, token_all_veh):
    data = data.astype(_F32)
    tok = token_all_veh.astype(_F32)

    P = tok[:, -1, :, :]
    p8 = P.reshape(N_TOK, 8).T.reshape(8, 64, 128)

    dpad = jnp.zeros((8, 128), _F32)
    dpad = dpad.at[0, :N_ST].set(data[:, 0])
    dpad = dpad.at[1, :N_ST].set(data[:, 1])
    dpad = dpad.at[2, :N_ST].set(data[:, 2])

    feat8, gpad = pl.pallas_call(
        _prep_body,
        out_shape=(jax.ShapeDtypeStruct((8, 64, 128), _F32),
                   jax.ShapeDtypeStruct((8, 128), _F32)),
    )(p8, dpad)

    feat16 = jnp.concatenate(
        [feat8.reshape(8, N_TOK).T, jnp.zeros((N_TOK, 8), _F32)], axis=1)
    g16 = jnp.concatenate(
        [gpad[:, :N_ST].T, jnp.zeros((N_ST, 8), _F32)], axis=1)
    p_soa = p8.reshape(8, N_TILE, TOK_PER_TILE).transpose(1, 0, 2)

    out = _sc_loop(p_soa, feat16, g16)

    gt_idx = out[:, 3].astype(jnp.int32)[None]
    gt_pos = jnp.stack([out[:, 0], out[:, 1]], axis=-1)[None]
    gt_head = out[:, 2][None]
    return (data[:, :2], data[:, -1], gt_idx, gt_pos, gt_head,
            gt_idx, gt_pos, gt_head)


# speedup vs baseline: 8.8195x; 1.0568x over previous
import functools

import jax
import jax.numpy as jnp
from jax import lax
from jax.experimental import pallas as pl
from jax.experimental.pallas import tpu as pltpu
from jax.experimental.pallas import tpu_sc as plsc

N_TOK = 8192
N_ST = 18
N_TILE = 16
TOK_PER_TILE = N_TOK // N_TILE
VREGS = TOK_PER_TILE // 16

_PI = 3.14159265358979323846
_F32 = jnp.float32


def _prep_body(p8_ref, dpad_ref, feat_ref, g_ref):
    p = p8_ref[...]
    x0, y0, x1, y1 = p[0], p[1], p[2], p[3]
    x2, y2, x3, y3 = p[4], p[5], p[6], p[7]
    mx = (x0 + x1 + x2 + x3) * 0.25
    my = (y0 + y1 + y2 + y3) * 0.25
    dx = x0 - x3
    dy = y0 - y3
    a = jnp.arctan2(dy, dx)
    ca = jnp.cos(a)
    sa = jnp.sin(a)
    z = jnp.zeros_like(mx)
    feat_ref[...] = jnp.stack([mx, my, a, ca, sa, z, z, z], axis=0)

    d = dpad_ref[...]
    x = d[0]
    y = d[1]
    th = d[2]
    hc = 0.5 * jnp.cos(th)
    hs = 0.5 * jnp.sin(th)
    lc = 4.8 * hc
    ls = 4.8 * hs
    wc = 2.0 * hc
    ws = 2.0 * hs
    g_ref[...] = jnp.stack(
        [x + lc - ws, y + ls + wc,
         x + lc + ws, y + ls - wc,
         x - lc + ws, y - ls - wc,
         x - lc - ws, y - ls + wc],
        axis=0)


def _rsqrt(x):
    i = lax.bitcast_convert_type(x, jnp.int32)
    i = jnp.int32(0x5F3759DF) - lax.shift_right_logical(i, 1)
    y = lax.bitcast_convert_type(i, _F32)
    xh = x * _F32(0.5)
    for _ in range(3):
        y = y * (_F32(1.5) - xh * y * y)
    return y


def _sc_body(p_hbm, feat_hbm, g_hbm, out_hbm,
             p_v, g_v, out_v, cand_v, stage_v, featrow_v, sh_min, sh_feat):
    cid = lax.axis_index("c")
    sid = lax.axis_index("s")

    pltpu.sync_copy(p_hbm.at[sid], p_v)
    pltpu.sync_copy(g_hbm, g_v)
    pltpu.sync_copy(feat_hbm.at[pl.ds(sid * TOK_PER_TILE, TOK_PER_TILE)],
                    sh_feat.at[pl.ds(sid * TOK_PER_TILE, TOK_PER_TILE)])
    plsc.subcore_barrier()

    ii = jax.lax.broadcasted_iota(jnp.int32, (16,), 0)
    lanes = ii

    def step(i, carry):
        tx, ty, c, s, h = carry
        grow = g_v[i]
        gpx = []
        gpy = []
        for k in range(4):
            gx = grow[2 * k] - tx
            gy = grow[2 * k + 1] - ty
            gpx.append(jnp.full((16,), c * gx + s * gy, _F32))
            gpy.append(jnp.full((16,), c * gy - s * gx, _F32))

        def vbody(v, bc):
            bd, bi = bc
            base = v * 16
            dist = jnp.zeros((16,), _F32)
            for k in range(4):
                px = p_v[2 * k, pl.ds(base, 16)]
                py = p_v[2 * k + 1, pl.ds(base, 16)]
                dxv = px - gpx[k]
                dyv = py - gpy[k]
                d2 = jnp.maximum(dxv * dxv + dyv * dyv, _F32(1e-12))
                dist = dist + d2 * _rsqrt(d2)
            iv = jnp.full((16,), sid * TOK_PER_TILE + base, jnp.int32) + lanes
            m = dist < bd
            bd = jnp.where(m, dist, bd)
            bi = jnp.where(m, iv, bi)
            return bd, bi

        bd0 = jnp.full((16,), _F32(3.0e38))
        bi0 = jnp.zeros((16,), jnp.int32)
        bd, bi = lax.fori_loop(0, VREGS, vbody, (bd0, bi0))

        md = jnp.min(bd)
        li = jnp.min(jnp.where(bd == md, bi, jnp.int32(1 << 30)))

        stage_v[...] = jnp.where(ii == 0, jnp.full((16,), md),
                                 jnp.full((16,), li.astype(_F32)))
        pltpu.sync_copy(stage_v, sh_min.at[i, sid])
        plsc.subcore_barrier()
        pltpu.sync_copy(sh_min.at[i], cand_v)

        crow = cand_v[0]
        gbd = crow[0]
        gbi = crow[1]
        for t in range(1, N_TILE):
            crow = cand_v[t]
            dtv = crow[0]
            itv = crow[1]
            better = dtv < gbd
            gbd = jnp.where(better, dtv, gbd)
            gbi = jnp.where(better, itv, gbi)
        gidx = gbi.astype(jnp.int32)

        pltpu.sync_copy(sh_feat.at[gidx], featrow_v)
        fr = featrow_v[...]
        fmx = fr[0]
        fmy = fr[1]
        fa = fr[2]
        fca = fr[3]
        fsa = fr[4]

        tx2 = tx + c * fmx - s * fmy
        ty2 = ty + s * fmx + c * fmy
        c2 = c * fca - s * fsa
        s2 = s * fca + c * fsa
        h2 = h + fa
        h2 = jnp.where(h2 > _F32(_PI), h2 - _F32(2 * _PI), h2)
        h2 = jnp.where(h2 <= _F32(-_PI), h2 + _F32(2 * _PI), h2)

        out_v[i] = jnp.where(
            ii == 0, jnp.full((16,), tx2),
            jnp.where(ii == 1, jnp.full((16,), ty2),
                      jnp.where(ii == 2, jnp.full((16,), h2),
                                jnp.full((16,), gbi))))
        return tx2, ty2, c2, s2, h2

    carry0 = (_F32(0.0), _F32(0.0), _F32(1.0), _F32(0.0), _F32(0.0))
    lax.fori_loop(0, N_ST, step, carry0)

    @pl.when(jnp.logical_and(cid == 0, sid == 0))
    def _():
        pltpu.sync_copy(out_v, out_hbm)


_sc_loop = functools.partial(
    pl.kernel,
    out_type=jax.ShapeDtypeStruct((N_ST, 16), _F32),
    mesh=plsc.VectorSubcoreMesh(core_axis_name="c", subcore_axis_name="s",
                                num_cores=1, num_subcores=N_TILE),
    compiler_params=pltpu.CompilerParams(needs_layout_passes=False),
    scratch_types=[
        pltpu.VMEM((8, TOK_PER_TILE), _F32),
        pltpu.VMEM((N_ST, 16), _F32),
        pltpu.VMEM((N_ST, 16), _F32),
        pltpu.VMEM((N_TILE, 16), _F32),
        pltpu.VMEM((16,), _F32),
        pltpu.VMEM((16,), _F32),
        pltpu.VMEM_SHARED((N_ST, N_TILE, 16), _F32),
        pltpu.VMEM_SHARED((N_TOK, 16), _F32),
    ],
)(_sc_body)


def kernel(data, token_all_veh):
    data = data.astype(_F32)
    tok = token_all_veh.astype(_F32)

    P = tok[:, -1, :, :]
    p8 = P.reshape(N_TOK, 8).T.reshape(8, 64, 128)

    dpad = jnp.zeros((8, 128), _F32)
    dpad = dpad.at[0, :N_ST].set(data[:, 0])
    dpad = dpad.at[1, :N_ST].set(data[:, 1])
    dpad = dpad.at[2, :N_ST].set(data[:, 2])

    feat8, gpad = pl.pallas_call(
        _prep_body,
        out_shape=(jax.ShapeDtypeStruct((8, 64, 128), _F32),
                   jax.ShapeDtypeStruct((8, 128), _F32)),
    )(p8, dpad)

    feat16 = jnp.concatenate(
        [feat8.reshape(8, N_TOK).T, jnp.zeros((N_TOK, 8), _F32)], axis=1)
    g16 = jnp.concatenate(
        [gpad[:, :N_ST].T, jnp.zeros((N_ST, 8), _F32)], axis=1)
    p_soa = p8.reshape(8, N_TILE, TOK_PER_TILE).transpose(1, 0, 2)

    out = _sc_loop(p_soa, feat16, g16)

    gt_idx = out[:, 3].astype(jnp.int32)[None]
    gt_pos = jnp.stack([out[:, 0], out[:, 1]], axis=-1)[None]
    gt_head = out[:, 2][None]
    return (data[:, :2], data[:, -1], gt_idx, gt_pos, gt_head,
            gt_idx, gt_pos, gt_head)


# speedup vs baseline: 10.1803x; 1.1543x over previous
import functools

import jax
import jax.numpy as jnp
from jax import lax
from jax.experimental import pallas as pl
from jax.experimental.pallas import tpu as pltpu
from jax.experimental.pallas import tpu_sc as plsc

N_TOK = 8192
N_ST = 18
N_TILE = 16
TOK_PER_TILE = N_TOK // N_TILE
VREGS = TOK_PER_TILE // 16

_PI = 3.14159265358979323846
_F32 = jnp.float32


def _prep_body(p8_ref, feat_ref):
    p = p8_ref[...]
    x0, y0, x1, y1 = p[0], p[1], p[2], p[3]
    x2, y2, x3, y3 = p[4], p[5], p[6], p[7]
    mx = (x0 + x1 + x2 + x3) * 0.25
    my = (y0 + y1 + y2 + y3) * 0.25
    dx = x0 - x3
    dy = y0 - y3
    a = jnp.arctan2(dy, dx)
    ca = jnp.cos(a)
    sa = jnp.sin(a)
    z = jnp.zeros_like(mx)
    feat_ref[...] = jnp.stack([mx, my, a, ca, sa, z, z, z], axis=0)


def _rsqrt(x):
    i = lax.bitcast_convert_type(x, jnp.int32)
    i = jnp.int32(0x5F3759DF) - lax.shift_right_logical(i, 1)
    y = lax.bitcast_convert_type(i, _F32)
    xh = x * _F32(0.5)
    for _ in range(3):
        y = y * (_F32(1.5) - xh * y * y)
    return y


def _sc_body(p_hbm, f_hbm, din_hbm, out_hbm,
             p_v, f_v, din_v, out_v, cand_v, stage_v, sh_min):
    cid = lax.axis_index("c")
    sid = lax.axis_index("s")

    tok0 = sid * TOK_PER_TILE
    pltpu.sync_copy(p_hbm.at[:, pl.ds(tok0, TOK_PER_TILE)], p_v)
    pltpu.sync_copy(f_hbm.at[:, pl.ds(tok0, TOK_PER_TILE)], f_v)
    pltpu.sync_copy(din_hbm, din_v)

    ii = lax.broadcasted_iota(jnp.int32, (16,), 0)
    frows = jnp.clip(ii - 2, 0, 4)

    def step(i, carry):
        tx, ty, c, s, h = carry
        drow = din_v[i]
        x = drow[0]
        y = drow[1]
        hc = _F32(0.5) * drow[2]
        hs = _F32(0.5) * drow[3]
        lc = _F32(4.8) * hc
        ls = _F32(4.8) * hs
        wc = _F32(2.0) * hc
        ws = _F32(2.0) * hs
        gxs = (x + lc - ws, x + lc + ws, x - lc + ws, x - lc - ws)
        gys = (y + ls + wc, y + ls - wc, y - ls - wc, y - ls + wc)
        gpx = []
        gpy = []
        for k in range(4):
            ux = gxs[k] - tx
            uy = gys[k] - ty
            gpx.append(jnp.full((16,), c * ux + s * uy, _F32))
            gpy.append(jnp.full((16,), c * uy - s * ux, _F32))

        def vbody(v, bc):
            bd, bi = bc
            base = v * 16
            dist = jnp.zeros((16,), _F32)
            for k in range(4):
                px = p_v[2 * k, pl.ds(base, 16)]
                py = p_v[2 * k + 1, pl.ds(base, 16)]
                dxv = px - gpx[k]
                dyv = py - gpy[k]
                d2 = jnp.maximum(dxv * dxv + dyv * dyv, _F32(1e-12))
                dist = dist + d2 * _rsqrt(d2)
            iv = jnp.full((16,), base, jnp.int32) + ii
            m = dist < bd
            bd = jnp.where(m, dist, bd)
            bi = jnp.where(m, iv, bi)
            return bd, bi

        bd0 = jnp.full((16,), _F32(3.0e38))
        bi0 = jnp.zeros((16,), jnp.int32)
        bd, bi = lax.fori_loop(0, VREGS, vbody, (bd0, bi0), unroll=4)

        md = jnp.min(bd)
        li = jnp.min(jnp.where(bd == md, bi, jnp.int32(1 << 30)))
        feats = plsc.load_gather(f_v, [frows, jnp.full((16,), li, jnp.int32)])
        lif = (li + tok0).astype(_F32)

        stage_v[...] = jnp.where(ii == 0, jnp.full((16,), md),
                                 jnp.where(ii == 1, jnp.full((16,), lif),
                                           feats))
        pltpu.sync_copy(stage_v, sh_min.at[i, sid])
        plsc.subcore_barrier()
        pltpu.sync_copy(sh_min.at[i], cand_v)

        best = cand_v[0]
        bds = best[0]
        for t in range(1, N_TILE):
            crow = cand_v[t]
            dtv = crow[0]
            better = dtv < bds
            bds = jnp.where(better, dtv, bds)
            best = jnp.where(better, crow, best)

        gif = best[1]
        fmx = best[2]
        fmy = best[3]
        fa = best[4]
        fca = best[5]
        fsa = best[6]

        tx2 = tx + c * fmx - s * fmy
        ty2 = ty + s * fmx + c * fmy
        c2 = c * fca - s * fsa
        s2 = s * fca + c * fsa
        h2 = h + fa
        h2 = jnp.where(h2 > _F32(_PI), h2 - _F32(2 * _PI), h2)
        h2 = jnp.where(h2 <= _F32(-_PI), h2 + _F32(2 * _PI), h2)

        out_v[i] = jnp.where(
            ii == 0, jnp.full((16,), tx2),
            jnp.where(ii == 1, jnp.full((16,), ty2),
                      jnp.where(ii == 2, jnp.full((16,), h2),
                                jnp.full((16,), gif))))
        return tx2, ty2, c2, s2, h2

    carry0 = (_F32(0.0), _F32(0.0), _F32(1.0), _F32(0.0), _F32(0.0))
    lax.fori_loop(0, N_ST, step, carry0)

    @pl.when(jnp.logical_and(cid == 0, sid == 0))
    def _():
        pltpu.sync_copy(out_v, out_hbm)


_sc_loop = functools.partial(
    pl.kernel,
    out_type=jax.ShapeDtypeStruct((N_ST, 16), _F32),
    mesh=plsc.VectorSubcoreMesh(core_axis_name="c", subcore_axis_name="s",
                                num_cores=1, num_subcores=N_TILE),
    compiler_params=pltpu.CompilerParams(needs_layout_passes=False),
    scratch_types=[
        pltpu.VMEM((8, TOK_PER_TILE), _F32),
        pltpu.VMEM((8, TOK_PER_TILE), _F32),
        pltpu.VMEM((N_ST, 16), _F32),
        pltpu.VMEM((N_ST, 16), _F32),
        pltpu.VMEM((N_TILE, 16), _F32),
        pltpu.VMEM((16,), _F32),
        pltpu.VMEM_SHARED((N_ST, N_TILE, 16), _F32),
    ],
)(_sc_body)


def kernel(data, token_all_veh):
    data = data.astype(_F32)
    tok = token_all_veh.astype(_F32)

    P = tok[:, -1, :, :]
    p8 = P.reshape(N_TOK, 8).T

    feat8 = pl.pallas_call(
        _prep_body,
        out_shape=jax.ShapeDtypeStruct((8, 64, 128), _F32),
    )(p8.reshape(8, 64, 128))

    cth = jnp.cos(data[:, 2:3])
    sth = jnp.sin(data[:, 2:3])
    din = jnp.concatenate(
        [data[:, :2], cth, sth, jnp.zeros((N_ST, 12), _F32)], axis=1)

    out = _sc_loop(p8, feat8.reshape(8, N_TOK), din)

    gt_idx = out[:, 3].astype(jnp.int32)[None]
    gt_pos = jnp.stack([out[:, 0], out[:, 1]], axis=-1)[None]
    gt_head = out[:, 2][None]
    return (data[:, :2], data[:, -1], gt_idx, gt_pos, gt_head,
            gt_idx, gt_pos, gt_head)
